# baseline (speedup 1.0000x reference)
import functools

import jax
import jax.numpy as jnp
from jax import lax
from jax.experimental import pallas as pl
from jax.experimental.pallas import tpu as pltpu
from jax.experimental.pallas import tpu_sc as plsc

BATCH = 4096
SEQ = 200
EMB = 64
LANES = 16

_NC = 2
_NS = 16
_NW = _NC * _NS
_SEQ_PER_W = BATCH // _NW
_SPLIT = 104
_REST = SEQ - _SPLIT


def _build():
    mesh = plsc.VectorSubcoreMesh(core_axis_name="c", subcore_axis_name="s")

    @functools.partial(
        pl.kernel,
        out_type=jax.ShapeDtypeStruct((BATCH, SEQ, EMB), jnp.float32),
        mesh=mesh,
        compiler_params=pltpu.CompilerParams(use_tc_tiling_on_sc=False),
        scratch_types=[
            pltpu.VMEM((SEQ,), jnp.int32),
            pltpu.VMEM((SEQ * EMB,), jnp.float32),
            pltpu.VMEM((SEQ, EMB), jnp.float32),
            pltpu.SemaphoreType.DMA,
            pltpu.SemaphoreType.DMA,
        ],
    )
    def k(x_hbm, table_hbm, pos_hbm, out_hbm,
          idx_v, pos_v, rows_v, sem_a, sem_b):
        wid = lax.axis_index("s") * _NC + lax.axis_index("c")
        pltpu.sync_copy(pos_hbm, pos_v)

        def body(s, carry):
            seq = wid * _SEQ_PER_W + s
            pltpu.sync_copy(x_hbm.at[pl.ds(seq * SEQ, SEQ)], idx_v)
            ca = pltpu.async_copy(
                table_hbm.at[idx_v.at[pl.ds(0, _SPLIT)]],
                rows_v.at[pl.ds(0, _SPLIT)], sem_a)
            cb = pltpu.async_copy(
                table_hbm.at[idx_v.at[pl.ds(_SPLIT, _REST)]],
                rows_v.at[pl.ds(_SPLIT, _REST)], sem_b)
            ca.wait()
            cb.wait()

            def add_row(i, c):
                for j in range(EMB // LANES):
                    sl = pl.ds(j * LANES, LANES)
                    rows_v[i, sl] = rows_v[i, sl] + pos_v[pl.ds(i * EMB + j * LANES, LANES)]
                return c

            lax.fori_loop(0, SEQ, add_row, 0)
            pltpu.sync_copy(rows_v, out_hbm.at[seq])
            return carry

        lax.fori_loop(0, _SEQ_PER_W, body, 0)

    return k


_KERNEL = _build()


def kernel(x, table, pos_encoding):
    x_flat = x.astype(jnp.int32).reshape(-1)
    pos_flat = pos_encoding[:SEQ].reshape(-1)
    return _KERNEL(x_flat, table, pos_flat)
